# Initial kernel scaffold; baseline (speedup 1.0000x reference)
import jax
import jax.numpy as jnp
from jax.experimental import pallas as pl


def kernel(indices, static_features, learnable_table):
    raise NotImplementedError("write your pallas kernel here")


# speedup vs baseline: 1.8581x; 1.8581x over previous
import functools

import jax
import jax.numpy as jnp
from jax import lax
from jax.experimental import pallas as pl
from jax.experimental.pallas import tpu as pltpu
from jax.experimental.pallas import tpu_sc as plsc

D = 32
LANES = 16
NW = 32
B = 16384
F = 26
BF = B * F
ROWS128 = BF // 128
PER_W_ROWS = ROWS128 // NW
SUB = 4
CHUNK = SUB * 128
N_CHUNKS = PER_W_ROWS // SUB


def _sc_gather_concat(static_features, learnable_table, idx2d):
    mesh = plsc.VectorSubcoreMesh(core_axis_name="c", subcore_axis_name="s")

    @functools.partial(
        pl.kernel,
        mesh=mesh,
        compiler_params=pltpu.CompilerParams(use_tc_tiling_on_sc=False),
        out_type=jax.ShapeDtypeStruct((2 * BF, D), jnp.float32),
        scratch_types=[
            pltpu.VMEM((SUB, 128), jnp.int32),
            pltpu.VMEM((SUB, 128), jnp.int32),
            pltpu.VMEM((SUB, 128), jnp.int32),
            pltpu.VMEM((CHUNK, D), jnp.float32),
            pltpu.VMEM((CHUNK, D), jnp.float32),
            pltpu.SemaphoreType.DMA,
            pltpu.SemaphoreType.DMA,
        ],
    )
    def k(stat_hbm, learn_hbm, idx_hbm, out_hbm,
          idx_v, ev_v, od_v, s_v, l_v, gsem, ssem):
        wid = lax.axis_index("s") * 2 + lax.axis_index("c")
        row0 = wid * PER_W_ROWS
        lane2 = lax.broadcasted_iota(jnp.int32, (LANES,), 0) * 2

        def body(g, carry):
            r = row0 + g * SUB
            pltpu.sync_copy(idx_hbm.at[pl.ds(r, SUB)], idx_v)
            base = r * 256
            for j in range(SUB):
                for t in range(128 // LANES):
                    v = lane2 + (base + 2 * (j * 128 + t * LANES))
                    ev_v[j, pl.ds(t * LANES, LANES)] = v
                    od_v[j, pl.ds(t * LANES, LANES)] = v + 1
            for j in range(SUB):
                pltpu.make_async_copy(
                    stat_hbm.at[idx_v.at[j]], s_v.at[pl.ds(j * 128, 128)], gsem).start()
                pltpu.make_async_copy(
                    learn_hbm.at[idx_v.at[j]], l_v.at[pl.ds(j * 128, 128)], gsem).start()
            for j in range(SUB):
                pltpu.make_async_copy(
                    stat_hbm.at[idx_v.at[j]], s_v.at[pl.ds(j * 128, 128)], gsem).wait()
                pltpu.make_async_copy(
                    learn_hbm.at[idx_v.at[j]], l_v.at[pl.ds(j * 128, 128)], gsem).wait()
            for j in range(SUB):
                pltpu.make_async_copy(
                    s_v.at[pl.ds(j * 128, 128)], out_hbm.at[ev_v.at[j]], ssem).start()
                pltpu.make_async_copy(
                    l_v.at[pl.ds(j * 128, 128)], out_hbm.at[od_v.at[j]], ssem).start()
            for j in range(SUB):
                pltpu.make_async_copy(
                    s_v.at[pl.ds(j * 128, 128)], out_hbm.at[ev_v.at[j]], ssem).wait()
                pltpu.make_async_copy(
                    l_v.at[pl.ds(j * 128, 128)], out_hbm.at[od_v.at[j]], ssem).wait()
            return carry

        lax.fori_loop(0, N_CHUNKS, body, 0)

    return k(static_features, learnable_table, idx2d)


def kernel(indices, static_features, learnable_table):
    idx2d = indices.astype(jnp.int32).reshape(ROWS128, 128)
    out2 = _sc_gather_concat(static_features, learnable_table, idx2d)
    return out2.reshape(B, F, 2 * D)


# speedup vs baseline: 1.9006x; 1.0228x over previous
import functools

import jax
import jax.numpy as jnp
from jax import lax
from jax.experimental import pallas as pl
from jax.experimental.pallas import tpu as pltpu
from jax.experimental.pallas import tpu_sc as plsc

D = 32
LANES = 16
NW = 32
B = 16384
F = 26
BF = B * F
ROWS128 = BF // 128
PER_W_ROWS = ROWS128 // NW
SUB = 4
CHUNK = SUB * 128
N_CHUNKS = PER_W_ROWS // SUB
NBUF = 2


def _sc_gather_concat(static_features, learnable_table, idx2d):
    mesh = plsc.VectorSubcoreMesh(core_axis_name="c", subcore_axis_name="s")

    @functools.partial(
        pl.kernel,
        mesh=mesh,
        compiler_params=pltpu.CompilerParams(use_tc_tiling_on_sc=False),
        out_type=jax.ShapeDtypeStruct((2 * BF, D), jnp.float32),
        scratch_types=[
            pltpu.VMEM((PER_W_ROWS, 128), jnp.int32),
            pltpu.VMEM((PER_W_ROWS, 128), jnp.int32),
            pltpu.VMEM((PER_W_ROWS, 128), jnp.int32),
            pltpu.VMEM((CHUNK, D), jnp.float32),
            pltpu.VMEM((CHUNK, D), jnp.float32),
            pltpu.VMEM((CHUNK, D), jnp.float32),
            pltpu.VMEM((CHUNK, D), jnp.float32),
            pltpu.SemaphoreType.DMA,
            pltpu.SemaphoreType.DMA,
        ],
    )
    def k(stat_hbm, learn_hbm, idx_hbm, out_hbm,
          idx_all, ev_all, od_all, s0, s1, l0, l1, gsem, ssem):
        sbufs = (s0, s1)
        lbufs = (l0, l1)
        wid = lax.axis_index("s") * 2 + lax.axis_index("c")
        row0 = wid * PER_W_ROWS
        pltpu.sync_copy(idx_hbm.at[pl.ds(row0, PER_W_ROWS)], idx_all)

        lane2 = lax.broadcasted_iota(jnp.int32, (LANES,), 0) * 2

        def build_row(r, carry):
            base = (row0 + r) * 256
            for t in range(128 // LANES):
                v = lane2 + (base + 2 * t * LANES)
                ev_all[r, pl.ds(t * LANES, LANES)] = v
                od_all[r, pl.ds(t * LANES, LANES)] = v + 1
            return carry

        lax.fori_loop(0, PER_W_ROWS, build_row, 0)

        def gather_copies(g, sb, lb):
            for j in range(SUB):
                row = g * SUB + j
                yield pltpu.make_async_copy(
                    stat_hbm.at[idx_all.at[row]],
                    sb.at[pl.ds(j * 128, 128)], gsem)
                yield pltpu.make_async_copy(
                    learn_hbm.at[idx_all.at[row]],
                    lb.at[pl.ds(j * 128, 128)], gsem)

        def scatter_copies(g, sb, lb):
            for j in range(SUB):
                row = g * SUB + j
                yield pltpu.make_async_copy(
                    sb.at[pl.ds(j * 128, 128)], out_hbm.at[ev_all.at[row]], ssem)
                yield pltpu.make_async_copy(
                    lb.at[pl.ds(j * 128, 128)], out_hbm.at[od_all.at[row]], ssem)

        for g in range(N_CHUNKS):
            b = g % NBUF
            if g >= NBUF:
                for c in scatter_copies(g - NBUF, sbufs[b], lbufs[b]):
                    c.wait()
            for c in gather_copies(g, sbufs[b], lbufs[b]):
                c.start()
            if g >= 1:
                pb = (g - 1) % NBUF
                for c in gather_copies(g - 1, sbufs[pb], lbufs[pb]):
                    c.wait()
                for c in scatter_copies(g - 1, sbufs[pb], lbufs[pb]):
                    c.start()
        lb_ = (N_CHUNKS - 1) % NBUF
        for c in gather_copies(N_CHUNKS - 1, sbufs[lb_], lbufs[lb_]):
            c.wait()
        for c in scatter_copies(N_CHUNKS - 1, sbufs[lb_], lbufs[lb_]):
            c.start()
        for g in range(N_CHUNKS - NBUF, N_CHUNKS):
            for c in scatter_copies(g, sbufs[g % NBUF], lbufs[g % NBUF]):
                c.wait()

    return k(static_features, learnable_table, idx2d)


def kernel(indices, static_features, learnable_table):
    idx2d = indices.astype(jnp.int32).reshape(ROWS128, 128)
    out2 = _sc_gather_concat(static_features, learnable_table, idx2d)
    return out2.reshape(B, F, 2 * D)
